# baseline (speedup 1.0000x reference)
import functools

import jax
import jax.numpy as jnp
from jax import lax
from jax.experimental import pallas as pl
from jax.experimental.pallas import tpu as pltpu
from jax.experimental.pallas import tpu_sc as plsc

B = 16384
D = 128
N_GENRE = 18
DENSE_IN = 768

NC = 2
NS = 16
NW = NC * NS
BPW = B // NW
CHUNK = 128
NCHUNK = BPW // CHUNK

CB = 2048
NCK = B // CB
NBUF = 3


def _sc_gather(id_idx2d, id_table):
    mesh = plsc.VectorSubcoreMesh(
        core_axis_name="c", subcore_axis_name="s", num_cores=NC, num_subcores=NS
    )

    @functools.partial(
        pl.kernel,
        out_type=jax.ShapeDtypeStruct((B, 4 * D), jnp.float32),
        mesh=mesh,
        scratch_types=[
            pltpu.VMEM((NCHUNK, CHUNK), jnp.int32),
            pltpu.VMEM((BPW, D), jnp.float32),
            pltpu.SemaphoreType.DMA,
            pltpu.SemaphoreType.DMA,
        ],
    )
    def body(id_hbm, idtab_hbm, out, idx_id, rows, gsem, wsem):
        wid = lax.axis_index("s") * NC + lax.axis_index("c")
        base = wid * BPW
        row_base = wid * NCHUNK
        half = BPW // 2

        pltpu.sync_copy(id_hbm.at[pl.ds(row_base, NCHUNK)], idx_id)

        def g_copy(j):
            return pltpu.make_async_copy(
                idtab_hbm.at[idx_id.at[j]],
                rows.at[pl.ds(j * CHUNK, CHUNK)], gsem,
            )

        def w_copy(h):
            return pltpu.make_async_copy(
                rows.at[pl.ds(h * half, half)],
                out.at[pl.ds(base + h * half, half), pl.ds(0, D)], wsem,
            )

        for j in range(NCHUNK):
            g_copy(j).start()
        g_copy(0).wait()
        g_copy(1).wait()
        w_copy(0).start()
        g_copy(2).wait()
        g_copy(3).wait()
        w_copy(1).start()
        w_copy(0).wait()
        w_copy(1).wait()

    return body(id_idx2d, id_table)


def _tc_body(alias_hbm, date_ref, g_ref, t_hbm, dtab_ref, gm_ref, w_ref,
             b_ref, out_hbm, tbuf, obuf, rsem, wsem):
    def t_read(c):
        s = c % NBUF
        return pltpu.make_async_copy(
            t_hbm.at[pl.ds(c * CB, CB)], tbuf.at[s], rsem.at[s]
        )

    def o_write(c):
        s = c % NBUF
        return pltpu.make_async_copy(
            obuf.at[s], out_hbm.at[pl.ds(c * CB, CB), pl.ds(D, 3 * D)],
            wsem.at[s],
        )

    for c in range(NBUF):
        t_read(c).start()

    lanes = lax.broadcasted_iota(jnp.int32, (CB, D), 1)
    for c in range(NCK):
        s = c % NBUF
        t_read(c).wait()
        if c >= NBUF:
            o_write(c - NBUF).wait()

        ob = obuf.at[s]
        date_blk = date_ref[pl.ds(c * CB, CB), :]
        one_hot = (date_blk == lanes).astype(jnp.float32)
        ob[:, 0:D] = jnp.dot(
            one_hot, dtab_ref[...], preferred_element_type=jnp.float32
        )
        g_blk = g_ref[pl.ds(c * CB, CB), :]
        ob[:, D:2 * D] = jnp.dot(
            g_blk, gm_ref[...], preferred_element_type=jnp.float32
        )
        ob[:, 2 * D:3 * D] = (
            jnp.dot(tbuf[s], w_ref[...], preferred_element_type=jnp.float32)
            + b_ref[...]
        )

        o_write(c).start()
        nxt = c + NBUF
        if nxt < NCK:
            t_read(nxt).start()

    for c in range(NCK - NBUF, NCK):
        o_write(c).wait()


def kernel(id, date, genres, title_embedding, id_table, date_table,
           genre_embedding_matrix, W_dense, b_dense):
    id2d = id.astype(jnp.int32).reshape(NW * NCHUNK, CHUNK)

    sc_out = _sc_gather(id2d, id_table)

    dtab_pad = jnp.zeros((D, D), jnp.float32).at[:100, :].set(date_table)

    out = pl.pallas_call(
        _tc_body,
        in_specs=[
            pl.BlockSpec(memory_space=pl.ANY),
            pl.BlockSpec(memory_space=pltpu.VMEM),
            pl.BlockSpec(memory_space=pltpu.VMEM),
            pl.BlockSpec(memory_space=pl.ANY),
            pl.BlockSpec(memory_space=pltpu.VMEM),
            pl.BlockSpec(memory_space=pltpu.VMEM),
            pl.BlockSpec(memory_space=pltpu.VMEM),
            pl.BlockSpec(memory_space=pltpu.VMEM),
        ],
        out_specs=pl.BlockSpec(memory_space=pl.ANY),
        out_shape=jax.ShapeDtypeStruct((B, 4 * D), jnp.float32),
        input_output_aliases={0: 0},
        scratch_shapes=[
            pltpu.VMEM((NBUF, CB, DENSE_IN), jnp.float32),
            pltpu.VMEM((NBUF, CB, 3 * D), jnp.float32),
            pltpu.SemaphoreType.DMA((NBUF,)),
            pltpu.SemaphoreType.DMA((NBUF,)),
        ],
    )(sc_out, date.astype(jnp.int32).reshape(B, 1), genres, title_embedding,
      dtab_pad, genre_embedding_matrix, W_dense, b_dense.reshape(1, D))
    return out


# speedup vs baseline: 1.0144x; 1.0144x over previous
import functools

import jax
import jax.numpy as jnp
from jax import lax
from jax.experimental import pallas as pl
from jax.experimental.pallas import tpu as pltpu
from jax.experimental.pallas import tpu_sc as plsc

B = 16384
D = 128
N_GENRE = 18
DENSE_IN = 768

NC = 2
NS = 16
NW = NC * NS
BPW = B // NW
CHUNK = 128
NCHUNK = BPW // CHUNK

CB = 1024
NCK = B // CB
NBUF = 6


def _sc_gather(id_idx2d, id_table):
    mesh = plsc.VectorSubcoreMesh(
        core_axis_name="c", subcore_axis_name="s", num_cores=NC, num_subcores=NS
    )

    @functools.partial(
        pl.kernel,
        out_type=jax.ShapeDtypeStruct((B, 4 * D), jnp.float32),
        mesh=mesh,
        scratch_types=[
            pltpu.VMEM((NCHUNK, CHUNK), jnp.int32),
            pltpu.VMEM((BPW, D), jnp.float32),
            pltpu.SemaphoreType.DMA,
            pltpu.SemaphoreType.DMA,
        ],
    )
    def body(id_hbm, idtab_hbm, out, idx_id, rows, gsem, wsem):
        wid = lax.axis_index("s") * NC + lax.axis_index("c")
        base = wid * BPW
        row_base = wid * NCHUNK
        half = BPW // 2

        pltpu.sync_copy(id_hbm.at[pl.ds(row_base, NCHUNK)], idx_id)

        def g_copy(j):
            return pltpu.make_async_copy(
                idtab_hbm.at[idx_id.at[j]],
                rows.at[pl.ds(j * CHUNK, CHUNK)], gsem,
            )

        def w_copy(h):
            return pltpu.make_async_copy(
                rows.at[pl.ds(h * half, half)],
                out.at[pl.ds(base + h * half, half), pl.ds(0, D)], wsem,
            )

        for j in range(NCHUNK):
            g_copy(j).start()
        g_copy(0).wait()
        g_copy(1).wait()
        w_copy(0).start()
        g_copy(2).wait()
        g_copy(3).wait()
        w_copy(1).start()
        w_copy(0).wait()
        w_copy(1).wait()

    return body(id_idx2d, id_table)


def _tc_body(alias_hbm, date_ref, g_ref, t_hbm, dtab_ref, gm_ref, w_ref,
             b_ref, out_hbm, tbuf, obuf, rsem, wsem):
    def t_read(c):
        s = c % NBUF
        return pltpu.make_async_copy(
            t_hbm.at[pl.ds(c * CB, CB)], tbuf.at[s], rsem.at[s]
        )

    def o_write(c):
        s = c % NBUF
        return pltpu.make_async_copy(
            obuf.at[s], out_hbm.at[pl.ds(c * CB, CB), pl.ds(D, 3 * D)],
            wsem.at[s],
        )

    for c in range(NBUF):
        t_read(c).start()

    lanes = lax.broadcasted_iota(jnp.int32, (CB, D), 1)
    for c in range(NCK):
        s = c % NBUF
        t_read(c).wait()
        if c >= NBUF:
            o_write(c - NBUF).wait()

        ob = obuf.at[s]
        date_blk = date_ref[pl.ds(c * CB, CB), :]
        one_hot = (date_blk == lanes).astype(jnp.float32)
        ob[:, 0:D] = jnp.dot(
            one_hot, dtab_ref[...], preferred_element_type=jnp.float32
        )
        g_blk = g_ref[pl.ds(c * CB, CB), :]
        ob[:, D:2 * D] = jnp.dot(
            g_blk, gm_ref[...], preferred_element_type=jnp.float32
        )
        ob[:, 2 * D:3 * D] = (
            jnp.dot(tbuf[s], w_ref[...], preferred_element_type=jnp.float32)
            + b_ref[...]
        )

        o_write(c).start()
        nxt = c + NBUF
        if nxt < NCK:
            t_read(nxt).start()

    for c in range(NCK - NBUF, NCK):
        o_write(c).wait()


def kernel(id, date, genres, title_embedding, id_table, date_table,
           genre_embedding_matrix, W_dense, b_dense):
    id2d = id.astype(jnp.int32).reshape(NW * NCHUNK, CHUNK)

    sc_out = _sc_gather(id2d, id_table)

    dtab_pad = jnp.zeros((D, D), jnp.float32).at[:100, :].set(date_table)

    out = pl.pallas_call(
        _tc_body,
        in_specs=[
            pl.BlockSpec(memory_space=pl.ANY),
            pl.BlockSpec(memory_space=pltpu.VMEM),
            pl.BlockSpec(memory_space=pltpu.VMEM),
            pl.BlockSpec(memory_space=pl.ANY),
            pl.BlockSpec(memory_space=pltpu.VMEM),
            pl.BlockSpec(memory_space=pltpu.VMEM),
            pl.BlockSpec(memory_space=pltpu.VMEM),
            pl.BlockSpec(memory_space=pltpu.VMEM),
        ],
        out_specs=pl.BlockSpec(memory_space=pl.ANY),
        out_shape=jax.ShapeDtypeStruct((B, 4 * D), jnp.float32),
        input_output_aliases={0: 0},
        scratch_shapes=[
            pltpu.VMEM((NBUF, CB, DENSE_IN), jnp.float32),
            pltpu.VMEM((NBUF, CB, 3 * D), jnp.float32),
            pltpu.SemaphoreType.DMA((NBUF,)),
            pltpu.SemaphoreType.DMA((NBUF,)),
        ],
    )(sc_out, date.astype(jnp.int32).reshape(B, 1), genres, title_embedding,
      dtab_pad, genre_embedding_matrix, W_dense, b_dense.reshape(1, D))
    return out
